# Initial kernel scaffold; baseline (speedup 1.0000x reference)
import jax
import jax.numpy as jnp
from jax.experimental import pallas as pl


def kernel(x, edge_index, node_graph_ids, desc_2d, desc_3d, W_gc1, b_gc1, W_gc2, b_gc2, W_pg, b_pg, W_p2, b_p2, W_att, W_fc1, b_fc1, W_fc2, b_fc2, W_fc3, b_fc3):
    raise NotImplementedError("write your pallas kernel here")


# speedup vs baseline: 6.5884x; 6.5884x over previous
import functools

import jax
import jax.numpy as jnp
from jax import lax
from jax.experimental import pallas as pl
from jax.experimental.pallas import tpu as pltpu
from jax.experimental.pallas import tpu_sc as plsc

N = 10000
E = 320000
B = 100
DIM_IN = 128
D1 = 100
DG = 20
D1P = 128
DGP = 32
D2D = 200
DH = 64
MLP1 = 128
MLP2 = 32
EPS = 1e-5

NTILES = 32
CH = 80
PER_TILE = E // NTILES
NIT = PER_TILE // CH
RC = 80
NRC = N // RC
RR = -(-NRC // 16)


def _proj1_body(x_ref, w_ref, o_ref):
    q = jnp.dot(x_ref[...], w_ref[...], preferred_element_type=jnp.float32)
    col = lax.broadcasted_iota(jnp.int32, q.shape, 1)
    o_ref[...] = jnp.where(col == D1, 1.0, q)


def _mid_body(p_ref, w_ref, b_ref, o_ref):
    acc = p_ref[0] + p_ref[1]
    deg = jnp.maximum(acc[:, D1:D1 + 1], 1.0)
    h1 = jnp.maximum(acc * (1.0 / deg) + b_ref[...], 0.0)
    q2 = jnp.dot(h1, w_ref[...], preferred_element_type=jnp.float32)
    col = lax.broadcasted_iota(jnp.int32, q2.shape, 1)
    o_ref[...] = jnp.where(col == DG, 1.0, q2)


def _tail_body(p_ref, ids_ref, b2_ref, d2_ref, wpg_ref, bpg_ref, wp2_ref,
               bp2_ref, watt_ref, w3a_ref, w3b_ref, bf1_ref, wf2_ref,
               bf2_ref, wf3_ref, bf3_ref, o_ref):
    acc = p_ref[0] + p_ref[1]
    deg = jnp.maximum(acc[:, DG:DG + 1], 1.0)
    h2 = jnp.maximum(acc * (1.0 / deg) + b2_ref[...], 0.0)
    col = lax.broadcasted_iota(jnp.int32, h2.shape, 1)
    h2 = jnp.where(col == DG, 1.0, h2)

    ids = ids_ref[...]
    gid = lax.broadcasted_iota(jnp.int32, (B, N), 0)
    member = (gid == ids).astype(jnp.float32)
    s = jnp.dot(member, h2, preferred_element_type=jnp.float32)
    cnt = jnp.maximum(s[:, DG:DG + 1], 1.0)
    hgf = s * (1.0 / cnt)

    h_g = jnp.dot(hgf, wpg_ref[...], preferred_element_type=jnp.float32) + bpg_ref[...]
    d2 = d2_ref[...]
    h_d = jnp.dot(d2, wp2_ref[...], preferred_element_type=jnp.float32) + bp2_ref[...]
    t = jnp.dot(h_g, watt_ref[...], preferred_element_type=jnp.float32)
    score = jnp.sum(t * h_d, axis=1, keepdims=True)
    a = 1.0 / (1.0 + jnp.exp(-score))
    g2 = a * d2

    tt = jnp.dot(g2, w3a_ref[...], preferred_element_type=jnp.float32) + w3b_ref[...]
    out1 = hgf[:, 0:1] * tt[:, 0:MLP1]
    for i in range(1, DG + 1):
        out1 = out1 + hgf[:, i:i + 1] * tt[:, i * MLP1:(i + 1) * MLP1]
    out1 = out1 + bf1_ref[...]
    mu = jnp.mean(out1, axis=0, keepdims=True)
    var = jnp.mean((out1 - mu) ** 2, axis=0, keepdims=True)
    out1 = jnp.maximum((out1 - mu) / jnp.sqrt(var + EPS), 0.0)

    out2 = jnp.dot(out1, wf2_ref[...], preferred_element_type=jnp.float32) + bf2_ref[...]
    mu2 = jnp.mean(out2, axis=0, keepdims=True)
    var2 = jnp.mean((out2 - mu2) ** 2, axis=0, keepdims=True)
    out2 = jnp.maximum((out2 - mu2) / jnp.sqrt(var2 + EPS), 0.0)

    o_ref[...] = jnp.dot(out2, wf3_ref[...], preferred_element_type=jnp.float32) + bf3_ref[...]


@functools.cache
def _make_agg(width):
    mesh = plsc.VectorSubcoreMesh(core_axis_name="c", subcore_axis_name="s")

    @functools.partial(
        pl.kernel,
        out_type=jax.ShapeDtypeStruct((2, N, width), jnp.float32),
        mesh=mesh,
        scratch_types=[
            pltpu.VMEM_SHARED((N, width), jnp.float32),
            pltpu.VMEM((RC, width), jnp.float32),
            pltpu.VMEM((CH,), jnp.int32),
            pltpu.VMEM((CH,), jnp.int32),
            pltpu.VMEM((CH, width), jnp.float32),
            pltpu.SemaphoreType.DMA,
        ],
        compiler_params=pltpu.CompilerParams(use_tc_tiling_on_sc=False),
    )
    def agg(q_hbm, src_hbm, dst_hbm, zer_hbm, out_hbm,
            acc_sh, buf, sidx, didx, rows, sem):
        c = lax.axis_index("c")
        s = lax.axis_index("s")
        wid = s * 2 + c
        pltpu.sync_copy(zer_hbm, buf)
        for j in range(RR):
            idx = s + 16 * j
            @pl.when(idx < NRC)
            def _():
                pltpu.sync_copy(buf, acc_sh.at[pl.ds(idx * RC, RC)])
        plsc.subcore_barrier()
        base = wid * PER_TILE
        def body(i, carry):
            off = base + i * CH
            pltpu.sync_copy(src_hbm.at[pl.ds(off, CH)], sidx)
            pltpu.async_copy(q_hbm.at[sidx], rows, sem).wait()
            pltpu.sync_copy(dst_hbm.at[pl.ds(off, CH)], didx)
            pltpu.sync_copy(rows, acc_sh.at[didx], add=True)
            return carry
        lax.fori_loop(0, NIT, body, 0)
        plsc.subcore_barrier()
        for j in range(RR):
            idx = s + 16 * j
            @pl.when(idx < NRC)
            def _():
                pltpu.sync_copy(acc_sh.at[pl.ds(idx * RC, RC)], buf)
                pltpu.sync_copy(buf, out_hbm.at[c, pl.ds(idx * RC, RC)])

    return agg


def kernel(x, edge_index, node_graph_ids, desc_2d, desc_3d,
           W_gc1, b_gc1, W_gc2, b_gc2, W_pg, b_pg, W_p2, b_p2, W_att,
           W_fc1, b_fc1, W_fc2, b_fc2, W_fc3, b_fc3):
    f32 = jnp.float32
    src = edge_index[0]
    dst = edge_index[1]
    ids2d = node_graph_ids.reshape(1, N)
    zer1 = jnp.zeros((RC, D1P), f32)
    zer2 = jnp.zeros((RC, DGP), f32)

    W1p = jnp.pad(W_gc1, ((0, 0), (0, D1P - D1)))
    b1p = jnp.pad(b_gc1, (0, D1P - D1)).reshape(1, D1P)
    W2p = jnp.pad(W_gc2, ((0, D1P - D1), (0, DGP - DG)))
    b2p = jnp.pad(b_gc2, (0, DGP - DG)).reshape(1, DGP)
    Wpgp = jnp.pad(W_pg, ((0, DGP - DG), (0, 0)))
    W3 = W_fc1.reshape(DG + 1, D2D + 1, MLP1).transpose(1, 0, 2)
    W3 = W3.reshape(D2D + 1, (DG + 1) * MLP1)
    W3a = W3[:D2D]
    W3b = W3[D2D:D2D + 1]

    q1 = pl.pallas_call(
        _proj1_body,
        out_shape=jax.ShapeDtypeStruct((N, D1P), f32),
    )(x, W1p)

    p1 = _make_agg(D1P)(q1, src, dst, zer1)

    q2 = pl.pallas_call(
        _mid_body,
        out_shape=jax.ShapeDtypeStruct((N, DGP), f32),
    )(p1, W2p, b1p)

    p2 = _make_agg(DGP)(q2, src, dst, zer2)

    out = pl.pallas_call(
        _tail_body,
        out_shape=jax.ShapeDtypeStruct((B, 1), f32),
    )(p2, ids2d, b2p, desc_2d, Wpgp, b_pg.reshape(1, DH),
      W_p2, b_p2.reshape(1, DH), W_att, W3a, W3b,
      b_fc1.reshape(1, MLP1), W_fc2, b_fc2.reshape(1, MLP2),
      W_fc3, b_fc3.reshape(1, 1))
    return out


# speedup vs baseline: 14.5201x; 2.2039x over previous
import functools

import jax
import jax.numpy as jnp
from jax import lax
from jax.experimental import pallas as pl
from jax.experimental.pallas import tpu as pltpu
from jax.experimental.pallas import tpu_sc as plsc

N = 10000
E = 320000
B = 100
DIM_IN = 128
D1 = 100
DG = 20
D1P = 128
DGP = 32
D2D = 200
DH = 64
MLP1 = 128
MLP2 = 32
EPS = 1e-5

NTILES = 32
CH = 80
PER_TILE = E // NTILES
NIT = PER_TILE // CH
RC = 80
NRC = N // RC
RR = -(-NRC // 16)


def _proj1_body(x_ref, w_ref, o_ref):
    q = jnp.dot(x_ref[...], w_ref[...], preferred_element_type=jnp.float32)
    col = lax.broadcasted_iota(jnp.int32, q.shape, 1)
    o_ref[...] = jnp.where(col == D1, 1.0, q)


def _mid_body(p_ref, w_ref, b_ref, o_ref):
    acc = p_ref[0] + p_ref[1]
    deg = jnp.maximum(acc[:, D1:D1 + 1], 1.0)
    h1 = jnp.maximum(acc * (1.0 / deg) + b_ref[...], 0.0)
    q2 = jnp.dot(h1, w_ref[...], preferred_element_type=jnp.float32)
    col = lax.broadcasted_iota(jnp.int32, q2.shape, 1)
    o_ref[...] = jnp.where(col == DG, 1.0, q2)


def _tail_body(p_ref, ids_ref, b2_ref, d2_ref, wpg_ref, bpg_ref, wp2_ref,
               bp2_ref, watt_ref, w3a_ref, w3b_ref, bf1_ref, wf2_ref,
               bf2_ref, wf3_ref, bf3_ref, o_ref):
    acc = p_ref[0] + p_ref[1]
    deg = jnp.maximum(acc[:, DG:DG + 1], 1.0)
    h2 = jnp.maximum(acc * (1.0 / deg) + b2_ref[...], 0.0)
    col = lax.broadcasted_iota(jnp.int32, h2.shape, 1)
    h2 = jnp.where(col == DG, 1.0, h2)

    ids = ids_ref[...]
    gid = lax.broadcasted_iota(jnp.int32, (B, N), 0)
    member = (gid == ids).astype(jnp.float32)
    s = jnp.dot(member, h2, preferred_element_type=jnp.float32)
    cnt = jnp.maximum(s[:, DG:DG + 1], 1.0)
    hgf = s * (1.0 / cnt)

    h_g = jnp.dot(hgf, wpg_ref[...], preferred_element_type=jnp.float32) + bpg_ref[...]
    d2 = d2_ref[...]
    h_d = jnp.dot(d2, wp2_ref[...], preferred_element_type=jnp.float32) + bp2_ref[...]
    t = jnp.dot(h_g, watt_ref[...], preferred_element_type=jnp.float32)
    score = jnp.sum(t * h_d, axis=1, keepdims=True)
    a = 1.0 / (1.0 + jnp.exp(-score))
    g2 = a * d2

    tt = jnp.dot(g2, w3a_ref[...], preferred_element_type=jnp.float32) + w3b_ref[...]
    out1 = hgf[:, 0:1] * tt[:, 0:MLP1]
    for i in range(1, DG + 1):
        out1 = out1 + hgf[:, i:i + 1] * tt[:, i * MLP1:(i + 1) * MLP1]
    out1 = out1 + bf1_ref[...]
    mu = jnp.mean(out1, axis=0, keepdims=True)
    var = jnp.mean((out1 - mu) ** 2, axis=0, keepdims=True)
    out1 = jnp.maximum((out1 - mu) / jnp.sqrt(var + EPS), 0.0)

    out2 = jnp.dot(out1, wf2_ref[...], preferred_element_type=jnp.float32) + bf2_ref[...]
    mu2 = jnp.mean(out2, axis=0, keepdims=True)
    var2 = jnp.mean((out2 - mu2) ** 2, axis=0, keepdims=True)
    out2 = jnp.maximum((out2 - mu2) / jnp.sqrt(var2 + EPS), 0.0)

    o_ref[...] = jnp.dot(out2, wf3_ref[...], preferred_element_type=jnp.float32) + bf3_ref[...]


@functools.cache
def _make_agg(width):
    mesh = plsc.VectorSubcoreMesh(core_axis_name="c", subcore_axis_name="s")

    @functools.partial(
        pl.kernel,
        out_type=jax.ShapeDtypeStruct((2, N, width), jnp.float32),
        mesh=mesh,
        scratch_types=[
            pltpu.VMEM_SHARED((N, width), jnp.float32),
            [pltpu.VMEM((CH,), jnp.int32) for _ in range(4)],
            [pltpu.VMEM((CH,), jnp.int32) for _ in range(4)],
            [pltpu.VMEM((CH, width), jnp.float32) for _ in range(4)],
            [pltpu.SemaphoreType.DMA for _ in range(4)],
            [pltpu.SemaphoreType.DMA for _ in range(4)],
            [pltpu.SemaphoreType.DMA for _ in range(4)],
            [pltpu.SemaphoreType.DMA for _ in range(4)],
        ],
        compiler_params=pltpu.CompilerParams(use_tc_tiling_on_sc=False),
    )
    def agg(q_hbm, src_hbm, dst_hbm, zer_hbm, out_hbm,
            acc_sh, sidx, didx, rows, ssem, dsem, gsem, scsem):
        c = lax.axis_index("c")
        s = lax.axis_index("s")
        wid = s * 2 + c
        buf = rows[0]
        pltpu.sync_copy(zer_hbm, buf)
        for j in range(RR):
            idx = s + 16 * j
            @pl.when(idx < NRC)
            def _():
                pltpu.sync_copy(buf, acc_sh.at[pl.ds(idx * RC, RC)])
        plsc.subcore_barrier()
        base = wid * PER_TILE

        def idx_load(k, b):
            off = base + k * CH
            pltpu.async_copy(src_hbm.at[pl.ds(off, CH)], sidx[b], ssem[b])
            pltpu.async_copy(dst_hbm.at[pl.ds(off, CH)], didx[b], dsem[b])

        def idx_wait(b):
            pltpu.make_async_copy(src_hbm.at[pl.ds(0, CH)], sidx[b], ssem[b]).wait()
            pltpu.make_async_copy(dst_hbm.at[pl.ds(0, CH)], didx[b], dsem[b]).wait()

        def gather_start(b):
            pltpu.async_copy(q_hbm.at[sidx[b]], rows[b], gsem[b])

        def gather_wait(b):
            pltpu.make_async_copy(q_hbm.at[sidx[b]], rows[b], gsem[b]).wait()

        def scat_start(b):
            pltpu.async_copy(rows[b], acc_sh.at[didx[b]], scsem[b], add=True)

        def scat_wait(b):
            pltpu.make_async_copy(rows[b], acc_sh.at[didx[b]], scsem[b]).wait()

        for k in range(2):
            idx_load(k, k)
        pltpu.make_async_copy(src_hbm.at[pl.ds(0, CH)], sidx[0], ssem[0]).wait()
        gather_start(0)

        def body(i, carry):
            for u in range(4):
                k = 4 * i + u
                b1 = (u + 1) % 4
                b2 = (u + 2) % 4
                pltpu.make_async_copy(
                    src_hbm.at[pl.ds(0, CH)], sidx[b1], ssem[b1]).wait()
                gather_start(b1)
                gather_wait(u)
                pltpu.make_async_copy(
                    dst_hbm.at[pl.ds(0, CH)], didx[u], dsem[u]).wait()
                scat_start(u)
                @pl.when(k >= 2)
                def _():
                    pltpu.make_async_copy(
                        rows[b2], acc_sh.at[didx[b2]], scsem[b2]).wait()
                @pl.when(k + 2 < NIT)
                def _():
                    idx_load(k + 2, b2)
            return carry

        lax.fori_loop(0, NIT // 4, body, 0)
        gather_wait(0)
        pltpu.make_async_copy(dst_hbm.at[pl.ds(0, CH)], didx[0], dsem[0]).wait()
        scat_start(0)
        for b in (0, 2, 3):
            scat_wait(b)
        plsc.subcore_barrier()
        for j in range(RR):
            idx = s + 16 * j
            @pl.when(idx < NRC)
            def _():
                pltpu.sync_copy(acc_sh.at[pl.ds(idx * RC, RC)], buf)
                pltpu.sync_copy(buf, out_hbm.at[c, pl.ds(idx * RC, RC)])

    return agg


def kernel(x, edge_index, node_graph_ids, desc_2d, desc_3d,
           W_gc1, b_gc1, W_gc2, b_gc2, W_pg, b_pg, W_p2, b_p2, W_att,
           W_fc1, b_fc1, W_fc2, b_fc2, W_fc3, b_fc3):
    f32 = jnp.float32
    src = edge_index[0]
    dst = edge_index[1]
    ids2d = node_graph_ids.reshape(1, N)
    zer1 = jnp.zeros((RC, D1P), f32)
    zer2 = jnp.zeros((RC, DGP), f32)

    W1p = jnp.pad(W_gc1, ((0, 0), (0, D1P - D1)))
    b1p = jnp.pad(b_gc1, (0, D1P - D1)).reshape(1, D1P)
    W2p = jnp.pad(W_gc2, ((0, D1P - D1), (0, DGP - DG)))
    b2p = jnp.pad(b_gc2, (0, DGP - DG)).reshape(1, DGP)
    Wpgp = jnp.pad(W_pg, ((0, DGP - DG), (0, 0)))
    W3 = W_fc1.reshape(DG + 1, D2D + 1, MLP1).transpose(1, 0, 2)
    W3 = W3.reshape(D2D + 1, (DG + 1) * MLP1)
    W3a = W3[:D2D]
    W3b = W3[D2D:D2D + 1]

    q1 = pl.pallas_call(
        _proj1_body,
        out_shape=jax.ShapeDtypeStruct((N, D1P), f32),
    )(x, W1p)

    p1 = _make_agg(D1P)(q1, src, dst, zer1)

    q2 = pl.pallas_call(
        _mid_body,
        out_shape=jax.ShapeDtypeStruct((N, DGP), f32),
    )(p1, W2p, b1p)

    p2 = _make_agg(DGP)(q2, src, dst, zer2)

    out = pl.pallas_call(
        _tail_body,
        out_shape=jax.ShapeDtypeStruct((B, 1), f32),
    )(p2, ids2d, b2p, desc_2d, Wpgp, b_pg.reshape(1, DH),
      W_p2, b_p2.reshape(1, DH), W_att, W3a, W3b,
      b_fc1.reshape(1, MLP1), W_fc2, b_fc2.reshape(1, MLP2),
      W_fc3, b_fc3.reshape(1, 1))
    return out
